# Initial kernel scaffold; baseline (speedup 1.0000x reference)
import jax
import jax.numpy as jnp
from jax.experimental import pallas as pl


def kernel(x_hero, x_enemy, x_bullet, x_door, x_wall, ei_defeats, ei_dodges, ei_to_go_to, ei_sees_block, W_defeats, b_defeats, W_dodges, b_dodges, W_to_go_to, b_to_go_to, W_sees_block, b_sees_block, W_fc, b_fc):
    raise NotImplementedError("write your pallas kernel here")


# speedup vs baseline: 194.2620x; 194.2620x over previous
import jax
import jax.numpy as jnp
from jax import lax
from jax.experimental import pallas as pl
from jax.experimental.pallas import tpu as pltpu
from jax.experimental.pallas import tpu_sc as plsc

N = 100000
E = 3200000
H = 128
OUT = 4

NC = 2
NS = 16
LN = 128

ROWS = 25088
EPAD = ROWS * LN
CK = 8
RPT = ROWS // NS
NCHUNK = RPT // CK

NPAD = 114688
SLICE = NPAD // NS
NV = SLICE // 16


def _sc_body(d0, d1, d2, d3, s0, s1, s2, s3,
             w0, w1, w2, w3,
             srcb, dstb, gathb, ones_v, degb, tb,
             deg_s, c_s, t_s, sem0, sem1):
    cid = lax.axis_index("c")
    sid = lax.axis_index("s")
    lo = sid * SLICE
    base0 = sid * RPT

    for v in range(8):
        ones_v[pl.ds(v * 16, 16)] = jnp.ones((16,), jnp.float32)

    def _rsqrt_inplace(buf):
        def _r(i, c):
            s = pl.ds(i * 16, 16)
            x = buf[s]
            bi = lax.bitcast_convert_type(x, jnp.int32)
            y = lax.bitcast_convert_type(
                jnp.int32(0x5F3759DF) - (bi >> 1), jnp.float32)
            y = y * (1.5 - 0.5 * x * y * y)
            y = y * (1.5 - 0.5 * x * y * y)
            y = y * (1.5 - 0.5 * x * y * y)
            buf[s] = jnp.where(x > 0.5, y, 0.0)
            return c
        lax.fori_loop(0, NV, _r, 0)

    def _run(dst_e, src_e, w_out):
        def _z(i, c):
            degb[pl.ds(i * 16, 16)] = jnp.zeros((16,), jnp.float32)
            return c
        lax.fori_loop(0, NV, _z, 0)
        pltpu.sync_copy(degb, deg_s.at[pl.ds(lo, SLICE)])
        pltpu.sync_copy(degb, c_s.at[pl.ds(lo, SLICE)])
        pltpu.sync_copy(degb, t_s.at[pl.ds(lo, SLICE)])
        plsc.subcore_barrier()

        def _a(i, c):
            rb = base0 + i * CK
            pltpu.sync_copy(dst_e.at[pl.ds(rb, CK), :], dstb)
            pltpu.sync_copy(src_e.at[pl.ds(rb, CK), :], srcb)
            descs = []
            for j in range(CK):
                descs.append(pltpu.async_copy(
                    ones_v, deg_s.at[dstb.at[j]], sem0, add=True))
                descs.append(pltpu.async_copy(
                    ones_v, c_s.at[srcb.at[j]], sem1, add=True))
            for dsc in descs:
                dsc.wait()
            return c
        lax.fori_loop(0, NCHUNK, _a, 0)
        plsc.subcore_barrier()

        pltpu.sync_copy(deg_s.at[pl.ds(lo, SLICE)], degb)
        _rsqrt_inplace(degb)
        pltpu.sync_copy(degb, deg_s.at[pl.ds(lo, SLICE)])
        plsc.subcore_barrier()

        def _b(i, c):
            rb = base0 + i * CK
            pltpu.sync_copy(dst_e.at[pl.ds(rb, CK), :], dstb)
            pltpu.sync_copy(src_e.at[pl.ds(rb, CK), :], srcb)
            gd = [pltpu.async_copy(deg_s.at[dstb.at[j]], gathb.at[j], sem0)
                  for j in range(CK)]
            for dsc in gd:
                dsc.wait()
            sd = [pltpu.async_copy(gathb.at[j], t_s.at[srcb.at[j]], sem1,
                                   add=True)
                  for j in range(CK)]
            for dsc in sd:
                dsc.wait()
            return c
        lax.fori_loop(0, NCHUNK, _b, 0)
        plsc.subcore_barrier()

        pltpu.sync_copy(c_s.at[pl.ds(lo, SLICE)], degb)
        _rsqrt_inplace(degb)
        pltpu.sync_copy(t_s.at[pl.ds(lo, SLICE)], tb)

        def _m(i, c):
            s = pl.ds(i * 16, 16)
            degb[s] = degb[s] * tb[s]
            return c
        lax.fori_loop(0, NV, _m, 0)
        pltpu.sync_copy(degb, w_out.at[pl.ds(lo, SLICE)])
        plsc.subcore_barrier()

    @pl.when(cid == 0)
    def _():
        _run(d0, s0, w0)
        _run(d1, s1, w1)

    @pl.when(cid == 1)
    def _():
        _run(d2, s2, w2)
        _run(d3, s3, w3)


@jax.jit
def _sc_call(d0, d1, d2, d3, s0, s1, s2, s3):
    f = pl.kernel(
        _sc_body,
        out_type=tuple(
            jax.ShapeDtypeStruct((NPAD,), jnp.float32) for _ in range(4)),
        mesh=plsc.VectorSubcoreMesh(core_axis_name="c", subcore_axis_name="s"),
        scratch_types=[
            pltpu.VMEM((CK, LN), jnp.int32),
            pltpu.VMEM((CK, LN), jnp.int32),
            pltpu.VMEM((CK, LN), jnp.float32),
            pltpu.VMEM((LN,), jnp.float32),
            pltpu.VMEM((SLICE,), jnp.float32),
            pltpu.VMEM((SLICE,), jnp.float32),
            pltpu.VMEM_SHARED((NPAD,), jnp.float32),
            pltpu.VMEM_SHARED((NPAD,), jnp.float32),
            pltpu.VMEM_SHARED((NPAD,), jnp.float32),
            pltpu.SemaphoreType.DMA,
            pltpu.SemaphoreType.DMA,
        ],
    )
    return f(d0, d1, d2, d3, s0, s1, s2, s3)


def _tc_body(w0, w1, w2, w3, xe, xb, xd, xw,
             wd, wdo, wt, ws_, bd, bdo, bt, bs, wfc, bfc, o_ref):
    bias = (bd[...] + bdo[...] + bt[...] + bs[...]).reshape(1, H)
    acc = jnp.zeros((1, H), jnp.float32)
    for wv, x, w in ((w0, xe, wd), (w1, xb, wdo), (w2, xd, wt), (w3, xw, ws_)):
        v = lax.dot_general(wv[...], x[...], (((1,), (1,)), ((), ())),
                            preferred_element_type=jnp.float32)
        acc = acc + jnp.dot(v, w[...], preferred_element_type=jnp.float32)
    hero_mean = acc * (1.0 / N) + bias
    o_ref[...] = (jnp.dot(hero_mean, wfc[...],
                          preferred_element_type=jnp.float32)
                  + bfc[...].reshape(1, OUT))


@jax.jit
def _tc_call(w0, w1, w2, w3, xe, xb, xd, xw,
             wd, wdo, wt, ws_, bd, bdo, bt, bs, wfc, bfc):
    return pl.pallas_call(
        _tc_body,
        out_shape=jax.ShapeDtypeStruct((1, OUT), jnp.float32),
    )(w0, w1, w2, w3, xe, xb, xd, xw, wd, wdo, wt, ws_, bd, bdo, bt, bs,
      wfc, bfc)


def _pad_split(ei):
    pad = (N + (jnp.arange(EPAD - E, dtype=jnp.int32) % (NPAD - N)))
    src = jnp.concatenate([ei[0], pad]).reshape(ROWS, LN)
    dst = jnp.concatenate([ei[1], pad]).reshape(ROWS, LN)
    return dst, src


def _pad_t(x):
    return jnp.pad(x.T, ((0, 0), (0, NPAD - N)))


def kernel(x_hero, x_enemy, x_bullet, x_door, x_wall,
           ei_defeats, ei_dodges, ei_to_go_to, ei_sees_block,
           W_defeats, b_defeats, W_dodges, b_dodges,
           W_to_go_to, b_to_go_to, W_sees_block, b_sees_block,
           W_fc, b_fc):
    d0, s0 = _pad_split(ei_defeats)
    d1, s1 = _pad_split(ei_dodges)
    d2, s2 = _pad_split(ei_to_go_to)
    d3, s3 = _pad_split(ei_sees_block)
    w0, w1, w2, w3 = _sc_call(d0, d1, d2, d3, s0, s1, s2, s3)
    return _tc_call(w0.reshape(1, NPAD), w1.reshape(1, NPAD),
                    w2.reshape(1, NPAD), w3.reshape(1, NPAD),
                    _pad_t(x_enemy), _pad_t(x_bullet),
                    _pad_t(x_door), _pad_t(x_wall),
                    W_defeats, W_dodges, W_to_go_to, W_sees_block,
                    b_defeats, b_dodges, b_to_go_to, b_sees_block,
                    W_fc, b_fc)


# speedup vs baseline: 310.4103x; 1.5979x over previous
import jax
import jax.numpy as jnp
from jax import lax
from jax.experimental import pallas as pl
from jax.experimental.pallas import tpu as pltpu
from jax.experimental.pallas import tpu_sc as plsc

N = 100000
E = 3200000
H = 128
OUT = 4

NC = 2
NS = 16
LN = 128

ROWS = 25088
EPAD = ROWS * LN
CK = 8
RPT = ROWS // NS
NCHUNK = RPT // CK
HALF = NCHUNK // 2

NPAD = 114688
SLICE = NPAD // NS
NV = SLICE // 16


def _sc_body(d0, d1, d2, d3, s0, s1, s2, s3,
             w0, w1, w2, w3,
             ib0, ib1, jb0, jb1, gb0, gb1, ones_v, degb, tb,
             deg_s, c_s, t_s,
             semi0, semi1, semg0, semg1, sems0, sems1):
    cid = lax.axis_index("c")
    sid = lax.axis_index("s")
    lo = sid * SLICE
    base0 = sid * RPT

    for v in range(8):
        ones_v[pl.ds(v * 16, 16)] = jnp.ones((16,), jnp.float32)

    def _rsqrt_inplace(buf):
        def _r(i, c):
            s = pl.ds(i * 16, 16)
            x = buf[s]
            bi = lax.bitcast_convert_type(x, jnp.int32)
            y = lax.bitcast_convert_type(
                jnp.int32(0x5F3759DF) - (bi >> 1), jnp.float32)
            y = y * (1.5 - 0.5 * x * y * y)
            y = y * (1.5 - 0.5 * x * y * y)
            y = y * (1.5 - 0.5 * x * y * y)
            buf[s] = jnp.where(x > 0.5, y, 0.0)
            return c
        lax.fori_loop(0, NV, _r, 0)

    def _run(dst_e, src_e, w_out):
        def _dr(sem):
            pltpu.make_async_copy(
                dst_e.at[pl.ds(base0, CK), :], ib0, sem).wait()

        def _ld(e_ref, i, buf, sem):
            return pltpu.async_copy(
                e_ref.at[pl.ds(base0 + i * CK, CK), :], buf, sem)

        def _z(i, c):
            degb[pl.ds(i * 16, 16)] = jnp.zeros((16,), jnp.float32)
            return c
        lax.fori_loop(0, NV, _z, 0)
        pltpu.sync_copy(degb, deg_s.at[pl.ds(lo, SLICE)])
        pltpu.sync_copy(degb, c_s.at[pl.ds(lo, SLICE)])
        pltpu.sync_copy(degb, t_s.at[pl.ds(lo, SLICE)])
        plsc.subcore_barrier()

        def _hist(e_ref, acc):
            _ld(e_ref, 0, ib0, semi0)

            def _h(g, c):
                @pl.when(g >= 1)
                def _():
                    _dr(semg1)
                _ld(e_ref, 2 * g + 1, ib1, semi1)
                _dr(semi0)
                for j in range(CK):
                    pltpu.async_copy(ones_v, acc.at[ib0.at[j]], semg0,
                                     add=True)
                _dr(semg0)
                @pl.when(g < HALF - 1)
                def _():
                    _ld(e_ref, 2 * g + 2, ib0, semi0)
                _dr(semi1)
                for j in range(CK):
                    pltpu.async_copy(ones_v, acc.at[ib1.at[j]], semg1,
                                     add=True)
                return c
            lax.fori_loop(0, HALF, _h, 0)
            _dr(semg1)
        _hist(dst_e, deg_s)
        _hist(src_e, c_s)
        plsc.subcore_barrier()

        pltpu.sync_copy(deg_s.at[pl.ds(lo, SLICE)], degb)
        _rsqrt_inplace(degb)
        pltpu.sync_copy(degb, deg_s.at[pl.ds(lo, SLICE)])
        plsc.subcore_barrier()

        _ld(dst_e, 0, ib0, semi0)
        _ld(src_e, 0, jb0, semi0)

        def _b(g, c):
            @pl.when(g >= 1)
            def _():
                _dr(sems1)
            _ld(dst_e, 2 * g + 1, ib1, semi1)
            _ld(src_e, 2 * g + 1, jb1, semi1)
            _dr(semi0)
            _dr(semi0)
            for j in range(CK):
                pltpu.async_copy(deg_s.at[ib0.at[j]], gb0.at[j], semg0)
            _dr(semg0)
            for j in range(CK):
                pltpu.async_copy(gb0.at[j], t_s.at[jb0.at[j]], sems0,
                                 add=True)
            _dr(sems0)
            @pl.when(g < HALF - 1)
            def _():
                _ld(dst_e, 2 * g + 2, ib0, semi0)
                _ld(src_e, 2 * g + 2, jb0, semi0)
            _dr(semi1)
            _dr(semi1)
            for j in range(CK):
                pltpu.async_copy(deg_s.at[ib1.at[j]], gb1.at[j], semg1)
            _dr(semg1)
            for j in range(CK):
                pltpu.async_copy(gb1.at[j], t_s.at[jb1.at[j]], sems1,
                                 add=True)
            return c
        lax.fori_loop(0, HALF, _b, 0)
        _dr(sems1)
        plsc.subcore_barrier()

        pltpu.sync_copy(c_s.at[pl.ds(lo, SLICE)], degb)
        _rsqrt_inplace(degb)
        pltpu.sync_copy(t_s.at[pl.ds(lo, SLICE)], tb)

        def _m(i, c):
            s = pl.ds(i * 16, 16)
            degb[s] = degb[s] * tb[s]
            return c
        lax.fori_loop(0, NV, _m, 0)
        pltpu.sync_copy(degb, w_out.at[pl.ds(lo, SLICE)])
        plsc.subcore_barrier()

    @pl.when(cid == 0)
    def _():
        _run(d0, s0, w0)
        _run(d1, s1, w1)

    @pl.when(cid == 1)
    def _():
        _run(d2, s2, w2)
        _run(d3, s3, w3)


@jax.jit
def _sc_call(d0, d1, d2, d3, s0, s1, s2, s3):
    f = pl.kernel(
        _sc_body,
        out_type=tuple(
            jax.ShapeDtypeStruct((NPAD,), jnp.float32) for _ in range(4)),
        mesh=plsc.VectorSubcoreMesh(core_axis_name="c", subcore_axis_name="s"),
        scratch_types=[
            pltpu.VMEM((CK, LN), jnp.int32),
            pltpu.VMEM((CK, LN), jnp.int32),
            pltpu.VMEM((CK, LN), jnp.int32),
            pltpu.VMEM((CK, LN), jnp.int32),
            pltpu.VMEM((CK, LN), jnp.float32),
            pltpu.VMEM((CK, LN), jnp.float32),
            pltpu.VMEM((LN,), jnp.float32),
            pltpu.VMEM((SLICE,), jnp.float32),
            pltpu.VMEM((SLICE,), jnp.float32),
            pltpu.VMEM_SHARED((NPAD,), jnp.float32),
            pltpu.VMEM_SHARED((NPAD,), jnp.float32),
            pltpu.VMEM_SHARED((NPAD,), jnp.float32),
            pltpu.SemaphoreType.DMA,
            pltpu.SemaphoreType.DMA,
            pltpu.SemaphoreType.DMA,
            pltpu.SemaphoreType.DMA,
            pltpu.SemaphoreType.DMA,
            pltpu.SemaphoreType.DMA,
        ],
    )
    return f(d0, d1, d2, d3, s0, s1, s2, s3)


def _tc_body(w0, w1, w2, w3, xe, xb, xd, xw,
             wd, wdo, wt, ws_, bd, bdo, bt, bs, wfc, bfc, o_ref):
    bias = (bd[...] + bdo[...] + bt[...] + bs[...]).reshape(1, H)
    acc = jnp.zeros((1, H), jnp.float32)
    for wv, x, w in ((w0, xe, wd), (w1, xb, wdo), (w2, xd, wt), (w3, xw, ws_)):
        v = lax.dot_general(wv[...], x[...], (((1,), (1,)), ((), ())),
                            preferred_element_type=jnp.float32)
        acc = acc + jnp.dot(v, w[...], preferred_element_type=jnp.float32)
    hero_mean = acc * (1.0 / N) + bias
    o_ref[...] = (jnp.dot(hero_mean, wfc[...],
                          preferred_element_type=jnp.float32)
                  + bfc[...].reshape(1, OUT))


@jax.jit
def _tc_call(w0, w1, w2, w3, xe, xb, xd, xw,
             wd, wdo, wt, ws_, bd, bdo, bt, bs, wfc, bfc):
    return pl.pallas_call(
        _tc_body,
        out_shape=jax.ShapeDtypeStruct((1, OUT), jnp.float32),
    )(w0, w1, w2, w3, xe, xb, xd, xw, wd, wdo, wt, ws_, bd, bdo, bt, bs,
      wfc, bfc)


def _pad_split(ei):
    pad = (N + (jnp.arange(EPAD - E, dtype=jnp.int32) % (NPAD - N)))
    src = jnp.concatenate([ei[0], pad]).reshape(ROWS, LN)
    dst = jnp.concatenate([ei[1], pad]).reshape(ROWS, LN)
    return dst, src


def _pad_t(x):
    return jnp.pad(x.T, ((0, 0), (0, NPAD - N)))


def kernel(x_hero, x_enemy, x_bullet, x_door, x_wall,
           ei_defeats, ei_dodges, ei_to_go_to, ei_sees_block,
           W_defeats, b_defeats, W_dodges, b_dodges,
           W_to_go_to, b_to_go_to, W_sees_block, b_sees_block,
           W_fc, b_fc):
    d0, s0 = _pad_split(ei_defeats)
    d1, s1 = _pad_split(ei_dodges)
    d2, s2 = _pad_split(ei_to_go_to)
    d3, s3 = _pad_split(ei_sees_block)
    w0, w1, w2, w3 = _sc_call(d0, d1, d2, d3, s0, s1, s2, s3)
    return _tc_call(w0.reshape(1, NPAD), w1.reshape(1, NPAD),
                    w2.reshape(1, NPAD), w3.reshape(1, NPAD),
                    _pad_t(x_enemy), _pad_t(x_bullet),
                    _pad_t(x_door), _pad_t(x_wall),
                    W_defeats, W_dodges, W_to_go_to, W_sees_block,
                    b_defeats, b_dodges, b_to_go_to, b_sees_block,
                    W_fc, b_fc)


# speedup vs baseline: 329.5221x; 1.0616x over previous
import jax
import jax.numpy as jnp
from jax import lax
from jax.experimental import pallas as pl
from jax.experimental.pallas import tpu as pltpu
from jax.experimental.pallas import tpu_sc as plsc

N = 100000
E = 3200000
H = 128
OUT = 4

NC = 2
NS = 16
LN = 128

RROWS = 25000
CK = 8
CHT = 196
LAST = 185
RPT = CHT * CK

NPAD = 114688
SLICE = NPAD // NS
NV = SLICE // 16


def _sc_body(d0, d1, d2, d3, s0, s1, s2, s3,
             w0, w1, w2, w3,
             ib0, ib1, jb0, jb1, gb0, gb1, ones_v, degb, tb,
             deg_s, c_s, t_s,
             semi0, semi1, semg0, semg1, sems0, sems1):
    cid = lax.axis_index("c")
    sid = lax.axis_index("s")
    lo = sid * SLICE
    base0 = sid * RPT

    for v in range(8):
        ones_v[pl.ds(v * 16, 16)] = jnp.ones((16,), jnp.float32)

    def _rsqrt_inplace(buf):
        def _r(i, c):
            s = pl.ds(i * 16, 16)
            x = buf[s]
            bi = lax.bitcast_convert_type(x, jnp.int32)
            y = lax.bitcast_convert_type(
                jnp.int32(0x5F3759DF) - (bi >> 1), jnp.float32)
            y = y * (1.5 - 0.5 * x * y * y)
            y = y * (1.5 - 0.5 * x * y * y)
            y = y * (1.5 - 0.5 * x * y * y)
            buf[s] = jnp.where(x > 0.5, y, 0.0)
            return c
        lax.fori_loop(0, NV, _r, 0)

    def _run(dst_e, src_e, w_out):
        def _dr(sem):
            pltpu.make_async_copy(
                dst_e.at[pl.ds(base0, CK), :], ib0, sem).wait()

        def _ld(e_ref, i, buf, sem):
            return pltpu.async_copy(
                e_ref.at[pl.ds(base0 + i * CK, CK), :], buf, sem)

        def _z(i, c):
            degb[pl.ds(i * 16, 16)] = jnp.zeros((16,), jnp.float32)
            return c
        lax.fori_loop(0, NV, _z, 0)
        pltpu.sync_copy(degb, deg_s.at[pl.ds(lo, SLICE)])
        pltpu.sync_copy(degb, c_s.at[pl.ds(lo, SLICE)])
        pltpu.sync_copy(degb, t_s.at[pl.ds(lo, SLICE)])
        plsc.subcore_barrier()

        half_t = jnp.where(sid == NS - 1, (LAST - 1) // 2, CHT // 2)

        def _hist(e_ref, acc):
            _ld(e_ref, 0, ib0, semi0)

            def _h(g, c):
                @pl.when(g >= 1)
                def _():
                    _dr(semg1)
                _ld(e_ref, 2 * g + 1, ib1, semi1)
                _dr(semi0)
                for j in range(CK):
                    pltpu.async_copy(ones_v, acc.at[ib0.at[j]], semg0,
                                     add=True)
                _dr(semg0)
                @pl.when(g < half_t - 1)
                def _():
                    _ld(e_ref, 2 * g + 2, ib0, semi0)
                _dr(semi1)
                for j in range(CK):
                    pltpu.async_copy(ones_v, acc.at[ib1.at[j]], semg1,
                                     add=True)
                return c
            lax.fori_loop(0, half_t, _h, 0)
            _dr(semg1)

            @pl.when(sid == NS - 1)
            def _():
                _ld(e_ref, LAST - 1, ib0, semi0)
                _dr(semi0)
                for j in range(CK):
                    pltpu.async_copy(ones_v, acc.at[ib0.at[j]], semg0,
                                     add=True)
                _dr(semg0)
        _hist(dst_e, deg_s)
        _hist(src_e, c_s)
        plsc.subcore_barrier()

        pltpu.sync_copy(deg_s.at[pl.ds(lo, SLICE)], degb)
        _rsqrt_inplace(degb)
        pltpu.sync_copy(degb, deg_s.at[pl.ds(lo, SLICE)])
        plsc.subcore_barrier()

        _ld(dst_e, 0, ib0, semi0)
        _ld(src_e, 0, jb0, semi0)

        def _b(g, c):
            @pl.when(g >= 1)
            def _():
                _dr(sems1)
            _ld(dst_e, 2 * g + 1, ib1, semi1)
            _ld(src_e, 2 * g + 1, jb1, semi1)
            _dr(semi0)
            _dr(semi0)
            for j in range(CK):
                pltpu.async_copy(deg_s.at[ib0.at[j]], gb0.at[j], semg0)
            _dr(semg0)
            for j in range(CK):
                pltpu.async_copy(gb0.at[j], t_s.at[jb0.at[j]], sems0,
                                 add=True)
            _dr(sems0)
            @pl.when(g < half_t - 1)
            def _():
                _ld(dst_e, 2 * g + 2, ib0, semi0)
                _ld(src_e, 2 * g + 2, jb0, semi0)
            _dr(semi1)
            _dr(semi1)
            for j in range(CK):
                pltpu.async_copy(deg_s.at[ib1.at[j]], gb1.at[j], semg1)
            _dr(semg1)
            for j in range(CK):
                pltpu.async_copy(gb1.at[j], t_s.at[jb1.at[j]], sems1,
                                 add=True)
            return c
        lax.fori_loop(0, half_t, _b, 0)
        _dr(sems1)

        @pl.when(sid == NS - 1)
        def _():
            _ld(dst_e, LAST - 1, ib0, semi0)
            _ld(src_e, LAST - 1, jb0, semi0)
            _dr(semi0)
            _dr(semi0)
            for j in range(CK):
                pltpu.async_copy(deg_s.at[ib0.at[j]], gb0.at[j], semg0)
            _dr(semg0)
            for j in range(CK):
                pltpu.async_copy(gb0.at[j], t_s.at[jb0.at[j]], sems0,
                                 add=True)
            _dr(sems0)
        plsc.subcore_barrier()

        pltpu.sync_copy(c_s.at[pl.ds(lo, SLICE)], degb)
        _rsqrt_inplace(degb)
        pltpu.sync_copy(t_s.at[pl.ds(lo, SLICE)], tb)

        def _m(i, c):
            s = pl.ds(i * 16, 16)
            degb[s] = degb[s] * tb[s]
            return c
        lax.fori_loop(0, NV, _m, 0)
        pltpu.sync_copy(degb, w_out.at[pl.ds(lo, SLICE)])
        plsc.subcore_barrier()

    @pl.when(cid == 0)
    def _():
        _run(d0, s0, w0)
        _run(d1, s1, w1)

    @pl.when(cid == 1)
    def _():
        _run(d2, s2, w2)
        _run(d3, s3, w3)


@jax.jit
def _sc_call(d0, d1, d2, d3, s0, s1, s2, s3):
    f = pl.kernel(
        _sc_body,
        out_type=tuple(
            jax.ShapeDtypeStruct((NPAD,), jnp.float32) for _ in range(4)),
        mesh=plsc.VectorSubcoreMesh(core_axis_name="c", subcore_axis_name="s"),
        scratch_types=[
            pltpu.VMEM((CK, LN), jnp.int32),
            pltpu.VMEM((CK, LN), jnp.int32),
            pltpu.VMEM((CK, LN), jnp.int32),
            pltpu.VMEM((CK, LN), jnp.int32),
            pltpu.VMEM((CK, LN), jnp.float32),
            pltpu.VMEM((CK, LN), jnp.float32),
            pltpu.VMEM((LN,), jnp.float32),
            pltpu.VMEM((SLICE,), jnp.float32),
            pltpu.VMEM((SLICE,), jnp.float32),
            pltpu.VMEM_SHARED((NPAD,), jnp.float32),
            pltpu.VMEM_SHARED((NPAD,), jnp.float32),
            pltpu.VMEM_SHARED((NPAD,), jnp.float32),
            pltpu.SemaphoreType.DMA,
            pltpu.SemaphoreType.DMA,
            pltpu.SemaphoreType.DMA,
            pltpu.SemaphoreType.DMA,
            pltpu.SemaphoreType.DMA,
            pltpu.SemaphoreType.DMA,
        ],
    )
    return f(d0, d1, d2, d3, s0, s1, s2, s3)


def _tc_body(w0, w1, w2, w3, xe, xb, xd, xw,
             wd, wdo, wt, ws_, bd, bdo, bt, bs, wfc, bfc, o_ref):
    bias = (bd[...] + bdo[...] + bt[...] + bs[...]).reshape(1, H)
    acc = jnp.zeros((1, H), jnp.float32)
    for wv, x, w in ((w0, xe, wd), (w1, xb, wdo), (w2, xd, wt), (w3, xw, ws_)):
        v = lax.dot_general(wv[...], x[...], (((1,), (1,)), ((), ())),
                            preferred_element_type=jnp.float32)
        acc = acc + jnp.dot(v, w[...], preferred_element_type=jnp.float32)
    hero_mean = acc * (1.0 / N) + bias
    o_ref[...] = (jnp.dot(hero_mean, wfc[...],
                          preferred_element_type=jnp.float32)
                  + bfc[...].reshape(1, OUT))


@jax.jit
def _tc_call(w0, w1, w2, w3, xe, xb, xd, xw,
             wd, wdo, wt, ws_, bd, bdo, bt, bs, wfc, bfc):
    return pl.pallas_call(
        _tc_body,
        out_shape=jax.ShapeDtypeStruct((1, OUT), jnp.float32),
    )(w0, w1, w2, w3, xe, xb, xd, xw, wd, wdo, wt, ws_, bd, bdo, bt, bs,
      wfc, bfc)


def _views(ei):
    return ei[1].reshape(RROWS, LN), ei[0].reshape(RROWS, LN)


def _pad_t(x):
    return jnp.pad(x.T, ((0, 0), (0, NPAD - N)))


def kernel(x_hero, x_enemy, x_bullet, x_door, x_wall,
           ei_defeats, ei_dodges, ei_to_go_to, ei_sees_block,
           W_defeats, b_defeats, W_dodges, b_dodges,
           W_to_go_to, b_to_go_to, W_sees_block, b_sees_block,
           W_fc, b_fc):
    d0, s0 = _views(ei_defeats)
    d1, s1 = _views(ei_dodges)
    d2, s2 = _views(ei_to_go_to)
    d3, s3 = _views(ei_sees_block)
    w0, w1, w2, w3 = _sc_call(d0, d1, d2, d3, s0, s1, s2, s3)
    return _tc_call(w0.reshape(1, NPAD), w1.reshape(1, NPAD),
                    w2.reshape(1, NPAD), w3.reshape(1, NPAD),
                    _pad_t(x_enemy), _pad_t(x_bullet),
                    _pad_t(x_door), _pad_t(x_wall),
                    W_defeats, W_dodges, W_to_go_to, W_sees_block,
                    b_defeats, b_dodges, b_to_go_to, b_sees_block,
                    W_fc, b_fc)


# speedup vs baseline: 366.3981x; 1.1119x over previous
import jax
import jax.numpy as jnp
from jax import lax
from jax.experimental import pallas as pl
from jax.experimental.pallas import tpu as pltpu
from jax.experimental.pallas import tpu_sc as plsc

N = 100000
E = 3200000
H = 128
OUT = 4

NC = 2
NS = 16
LN = 128

RROWS = 25000
CK = 8
CKH = 16
CHT = 196
LAST = 185
RPT = CHT * CK

NPAD = 114688
SLICE = NPAD // NS
NV = SLICE // 16


def _sc_body(d0, d1, d2, d3, s0, s1, s2, s3,
             w0, w1, w2, w3,
             ib0, ib1, jb0, jb1, gb0, gb1, hb0, hb1, ones_v, degb, tb,
             deg_s, c_s, t_s,
             semi0, semi1, semg0, semg1, sems0, sems1):
    cid = lax.axis_index("c")
    sid = lax.axis_index("s")
    lo = sid * SLICE
    base0 = sid * RPT

    for v in range(8):
        ones_v[pl.ds(v * 16, 16)] = jnp.ones((16,), jnp.float32)

    def _rsqrt_inplace(buf):
        def _r(i, c):
            s = pl.ds(i * 16, 16)
            x = buf[s]
            bi = lax.bitcast_convert_type(x, jnp.int32)
            y = lax.bitcast_convert_type(
                jnp.int32(0x5F3759DF) - (bi >> 1), jnp.float32)
            y = y * (1.5 - 0.5 * x * y * y)
            y = y * (1.5 - 0.5 * x * y * y)
            y = y * (1.5 - 0.5 * x * y * y)
            buf[s] = jnp.where(x > 0.5, y, 0.0)
            return c
        lax.fori_loop(0, NV, _r, 0)

    def _run(dst_e, src_e, w_out):
        def _dr(sem):
            pltpu.make_async_copy(
                dst_e.at[pl.ds(base0, CK), :], ib0, sem).wait()

        def _ld(e_ref, i, buf, sem):
            return pltpu.async_copy(
                e_ref.at[pl.ds(base0 + i * CK, CK), :], buf, sem)

        def _z(i, c):
            degb[pl.ds(i * 16, 16)] = jnp.zeros((16,), jnp.float32)
            return c
        lax.fori_loop(0, NV, _z, 0)
        pltpu.sync_copy(degb, deg_s.at[pl.ds(lo, SLICE)])
        pltpu.sync_copy(degb, c_s.at[pl.ds(lo, SLICE)])
        pltpu.sync_copy(degb, t_s.at[pl.ds(lo, SLICE)])
        plsc.subcore_barrier()

        half_t = jnp.where(sid == NS - 1, (LAST - 1) // 2, CHT // 2)

        def _hist(e_ref, acc):
            def _lh(i, buf, sem):
                return pltpu.async_copy(
                    e_ref.at[pl.ds(base0 + i * CKH, CKH), :], buf, sem)

            half_h = jnp.where(sid == NS - 1, 46, 49)
            _lh(0, hb0, semi0)

            def _h(g, c):
                @pl.when(g >= 1)
                def _():
                    _dr(semg1)
                    _dr(semg1)
                _lh(2 * g + 1, hb1, semi1)
                _dr(semi0)
                _dr(semi0)
                for j in range(CKH):
                    pltpu.async_copy(ones_v, acc.at[hb0.at[j]], semg0,
                                     add=True)
                _dr(semg0)
                _dr(semg0)
                @pl.when(g < half_h - 1)
                def _():
                    _lh(2 * g + 2, hb0, semi0)
                _dr(semi1)
                _dr(semi1)
                for j in range(CKH):
                    pltpu.async_copy(ones_v, acc.at[hb1.at[j]], semg1,
                                     add=True)
                return c
            lax.fori_loop(0, half_h, _h, 0)
            _dr(semg1)
            _dr(semg1)

            @pl.when(sid == NS - 1)
            def _():
                pltpu.async_copy(
                    e_ref.at[pl.ds(base0 + 92 * CKH, CK), :],
                    hb0.at[pl.ds(0, CK)], semi0)
                _dr(semi0)
                for j in range(CK):
                    pltpu.async_copy(ones_v, acc.at[hb0.at[j]], semg0,
                                     add=True)
                _dr(semg0)
        _hist(dst_e, deg_s)
        _hist(src_e, c_s)
        plsc.subcore_barrier()

        pltpu.sync_copy(deg_s.at[pl.ds(lo, SLICE)], degb)
        _rsqrt_inplace(degb)
        pltpu.sync_copy(degb, deg_s.at[pl.ds(lo, SLICE)])
        plsc.subcore_barrier()

        _ld(dst_e, 0, ib0, semi0)
        _ld(src_e, 0, jb0, semi0)

        def _b(g, c):
            @pl.when(g >= 1)
            def _():
                _dr(sems1)
            _ld(dst_e, 2 * g + 1, ib1, semi1)
            _ld(src_e, 2 * g + 1, jb1, semi1)
            _dr(semi0)
            _dr(semi0)
            for j in range(CK):
                pltpu.async_copy(deg_s.at[ib0.at[j]], gb0.at[j], semg0)
            _dr(semg0)
            for j in range(CK):
                pltpu.async_copy(gb0.at[j], t_s.at[jb0.at[j]], sems0,
                                 add=True)
            _dr(sems0)
            @pl.when(g < half_t - 1)
            def _():
                _ld(dst_e, 2 * g + 2, ib0, semi0)
                _ld(src_e, 2 * g + 2, jb0, semi0)
            _dr(semi1)
            _dr(semi1)
            for j in range(CK):
                pltpu.async_copy(deg_s.at[ib1.at[j]], gb1.at[j], semg1)
            _dr(semg1)
            for j in range(CK):
                pltpu.async_copy(gb1.at[j], t_s.at[jb1.at[j]], sems1,
                                 add=True)
            return c
        lax.fori_loop(0, half_t, _b, 0)
        _dr(sems1)

        @pl.when(sid == NS - 1)
        def _():
            _ld(dst_e, LAST - 1, ib0, semi0)
            _ld(src_e, LAST - 1, jb0, semi0)
            _dr(semi0)
            _dr(semi0)
            for j in range(CK):
                pltpu.async_copy(deg_s.at[ib0.at[j]], gb0.at[j], semg0)
            _dr(semg0)
            for j in range(CK):
                pltpu.async_copy(gb0.at[j], t_s.at[jb0.at[j]], sems0,
                                 add=True)
            _dr(sems0)
        plsc.subcore_barrier()

        pltpu.sync_copy(c_s.at[pl.ds(lo, SLICE)], degb)
        _rsqrt_inplace(degb)
        pltpu.sync_copy(t_s.at[pl.ds(lo, SLICE)], tb)

        def _m(i, c):
            s = pl.ds(i * 16, 16)
            degb[s] = degb[s] * tb[s]
            return c
        lax.fori_loop(0, NV, _m, 0)
        pltpu.sync_copy(degb, w_out.at[pl.ds(lo, SLICE)])
        plsc.subcore_barrier()

    @pl.when(cid == 0)
    def _():
        _run(d0, s0, w0)
        _run(d1, s1, w1)

    @pl.when(cid == 1)
    def _():
        _run(d2, s2, w2)
        _run(d3, s3, w3)


@jax.jit
def _sc_call(d0, d1, d2, d3, s0, s1, s2, s3):
    f = pl.kernel(
        _sc_body,
        out_type=tuple(
            jax.ShapeDtypeStruct((NPAD,), jnp.float32) for _ in range(4)),
        mesh=plsc.VectorSubcoreMesh(core_axis_name="c", subcore_axis_name="s"),
        scratch_types=[
            pltpu.VMEM((CK, LN), jnp.int32),
            pltpu.VMEM((CK, LN), jnp.int32),
            pltpu.VMEM((CK, LN), jnp.int32),
            pltpu.VMEM((CK, LN), jnp.int32),
            pltpu.VMEM((CK, LN), jnp.float32),
            pltpu.VMEM((CK, LN), jnp.float32),
            pltpu.VMEM((CKH, LN), jnp.int32),
            pltpu.VMEM((CKH, LN), jnp.int32),
            pltpu.VMEM((LN,), jnp.float32),
            pltpu.VMEM((SLICE,), jnp.float32),
            pltpu.VMEM((SLICE,), jnp.float32),
            pltpu.VMEM_SHARED((NPAD,), jnp.float32),
            pltpu.VMEM_SHARED((NPAD,), jnp.float32),
            pltpu.VMEM_SHARED((NPAD,), jnp.float32),
            pltpu.SemaphoreType.DMA,
            pltpu.SemaphoreType.DMA,
            pltpu.SemaphoreType.DMA,
            pltpu.SemaphoreType.DMA,
            pltpu.SemaphoreType.DMA,
            pltpu.SemaphoreType.DMA,
        ],
    )
    return f(d0, d1, d2, d3, s0, s1, s2, s3)


def _tc_body(w0, w1, w2, w3, xe, xb, xd, xw,
             wd, wdo, wt, ws_, bd, bdo, bt, bs, wfc, bfc, o_ref):
    bias = (bd[...] + bdo[...] + bt[...] + bs[...]).reshape(1, H)
    acc = jnp.zeros((1, H), jnp.float32)
    for wv, x, w in ((w0, xe, wd), (w1, xb, wdo), (w2, xd, wt), (w3, xw, ws_)):
        v = lax.dot_general(wv[...], x[...], (((1,), (1,)), ((), ())),
                            preferred_element_type=jnp.float32)
        acc = acc + jnp.dot(v, w[...], preferred_element_type=jnp.float32)
    hero_mean = acc * (1.0 / N) + bias
    o_ref[...] = (jnp.dot(hero_mean, wfc[...],
                          preferred_element_type=jnp.float32)
                  + bfc[...].reshape(1, OUT))


@jax.jit
def _tc_call(w0, w1, w2, w3, xe, xb, xd, xw,
             wd, wdo, wt, ws_, bd, bdo, bt, bs, wfc, bfc):
    return pl.pallas_call(
        _tc_body,
        out_shape=jax.ShapeDtypeStruct((1, OUT), jnp.float32),
    )(w0, w1, w2, w3, xe, xb, xd, xw, wd, wdo, wt, ws_, bd, bdo, bt, bs,
      wfc, bfc)


def _views(ei):
    return ei[1].reshape(RROWS, LN), ei[0].reshape(RROWS, LN)


def _pad_t(x):
    return jnp.pad(x.T, ((0, 0), (0, NPAD - N)))


def kernel(x_hero, x_enemy, x_bullet, x_door, x_wall,
           ei_defeats, ei_dodges, ei_to_go_to, ei_sees_block,
           W_defeats, b_defeats, W_dodges, b_dodges,
           W_to_go_to, b_to_go_to, W_sees_block, b_sees_block,
           W_fc, b_fc):
    d0, s0 = _views(ei_defeats)
    d1, s1 = _views(ei_dodges)
    d2, s2 = _views(ei_to_go_to)
    d3, s3 = _views(ei_sees_block)
    w0, w1, w2, w3 = _sc_call(d0, d1, d2, d3, s0, s1, s2, s3)
    return _tc_call(w0.reshape(1, NPAD), w1.reshape(1, NPAD),
                    w2.reshape(1, NPAD), w3.reshape(1, NPAD),
                    _pad_t(x_enemy), _pad_t(x_bullet),
                    _pad_t(x_door), _pad_t(x_wall),
                    W_defeats, W_dodges, W_to_go_to, W_sees_block,
                    b_defeats, b_dodges, b_to_go_to, b_sees_block,
                    W_fc, b_fc)


# speedup vs baseline: 406.1501x; 1.1085x over previous
import jax
import jax.numpy as jnp
from jax import lax
from jax.experimental import pallas as pl
from jax.experimental.pallas import tpu as pltpu
from jax.experimental.pallas import tpu_sc as plsc

N = 100000
E = 3200000
H = 128
OUT = 4

NC = 2
NS = 16
LN = 128

RROWS = 25000
CK = 8
CKH = 16
CHT = 196
LAST = 185
RPT = CHT * CK

NPAD = 114688
SLICE = NPAD // NS
NV = SLICE // 16


def _sc_body(d0, d1, d2, d3, s0, s1, s2, s3,
             w0, w1, w2, w3,
             ib0, ib1, jb0, jb1, gb0, gb1, hb0, hb1, ones_v, degb, tb,
             deg_s, c_s, t_s,
             semi0, semi1, semg0, semg1, sems0, sems1):
    cid = lax.axis_index("c")
    sid = lax.axis_index("s")
    lo = sid * SLICE
    base0 = sid * RPT

    for v in range(8):
        ones_v[pl.ds(v * 16, 16)] = jnp.ones((16,), jnp.float32)

    def _rsqrt_inplace(buf):
        def _r(i, c):
            s = pl.ds(i * 16, 16)
            x = buf[s]
            bi = lax.bitcast_convert_type(x, jnp.int32)
            y = lax.bitcast_convert_type(
                jnp.int32(0x5F3759DF) - (bi >> 1), jnp.float32)
            y = y * (1.5 - 0.5 * x * y * y)
            y = y * (1.5 - 0.5 * x * y * y)
            y = y * (1.5 - 0.5 * x * y * y)
            buf[s] = jnp.where(x > 0.5, y, 0.0)
            return c
        lax.fori_loop(0, NV, _r, 0)

    def _run(dst_e, src_e, w_out):
        def _dr(sem):
            pltpu.make_async_copy(
                dst_e.at[pl.ds(base0, CK), :], ib0, sem).wait()

        def _ld(e_ref, i, buf, sem):
            return pltpu.async_copy(
                e_ref.at[pl.ds(base0 + i * CK, CK), :], buf, sem)

        def _z(i, c):
            degb[pl.ds(i * 16, 16)] = jnp.zeros((16,), jnp.float32)
            return c
        lax.fori_loop(0, NV, _z, 0)
        pltpu.sync_copy(degb, deg_s.at[pl.ds(lo, SLICE)])
        pltpu.sync_copy(degb, c_s.at[pl.ds(lo, SLICE)])
        pltpu.sync_copy(degb, t_s.at[pl.ds(lo, SLICE)])
        plsc.subcore_barrier()

        half_t = jnp.where(sid == NS - 1, (LAST - 1) // 2, CHT // 2)

        def _hist(e_ref, acc):
            def _lh(i, buf, sem):
                return pltpu.async_copy(
                    e_ref.at[pl.ds(base0 + i * CKH, CKH), :], buf, sem)

            half_h = jnp.where(sid == NS - 1, 46, 49)
            _lh(0, hb0, semi0)

            def _h(g, c):
                _dr(semi0)
                _dr(semi0)
                for j in range(CKH):
                    pltpu.async_copy(ones_v, acc.at[hb0.at[j]], semg0,
                                     add=True)
                @pl.when(g >= 1)
                def _():
                    _dr(semg1)
                    _dr(semg1)
                _lh(2 * g + 1, hb1, semi1)
                _dr(semi1)
                _dr(semi1)
                for j in range(CKH):
                    pltpu.async_copy(ones_v, acc.at[hb1.at[j]], semg1,
                                     add=True)
                _dr(semg0)
                _dr(semg0)
                @pl.when(g < half_h - 1)
                def _():
                    _lh(2 * g + 2, hb0, semi0)
                return c
            lax.fori_loop(0, half_h, _h, 0)
            _dr(semg1)
            _dr(semg1)

            @pl.when(sid == NS - 1)
            def _():
                pltpu.async_copy(
                    e_ref.at[pl.ds(base0 + 92 * CKH, CK), :],
                    hb0.at[pl.ds(0, CK)], semi0)
                _dr(semi0)
                for j in range(CK):
                    pltpu.async_copy(ones_v, acc.at[hb0.at[j]], semg0,
                                     add=True)
                _dr(semg0)
        _hist(dst_e, deg_s)
        _hist(src_e, c_s)
        plsc.subcore_barrier()

        pltpu.sync_copy(deg_s.at[pl.ds(lo, SLICE)], degb)
        _rsqrt_inplace(degb)
        pltpu.sync_copy(degb, deg_s.at[pl.ds(lo, SLICE)])
        plsc.subcore_barrier()

        _ld(dst_e, 0, ib0, semi0)
        _ld(src_e, 0, jb0, semi0)

        def _b(g, c):
            _dr(semi0)
            _dr(semi0)
            for j in range(CK):
                pltpu.async_copy(deg_s.at[ib0.at[j]], gb0.at[j], semg0)
            @pl.when(g >= 1)
            def _():
                _dr(sems1)
            _ld(dst_e, 2 * g + 1, ib1, semi1)
            _ld(src_e, 2 * g + 1, jb1, semi1)
            _dr(semg0)
            for j in range(CK):
                pltpu.async_copy(gb0.at[j], t_s.at[jb0.at[j]], sems0,
                                 add=True)
            _dr(semi1)
            _dr(semi1)
            for j in range(CK):
                pltpu.async_copy(deg_s.at[ib1.at[j]], gb1.at[j], semg1)
            _dr(sems0)
            @pl.when(g < half_t - 1)
            def _():
                _ld(dst_e, 2 * g + 2, ib0, semi0)
                _ld(src_e, 2 * g + 2, jb0, semi0)
            _dr(semg1)
            for j in range(CK):
                pltpu.async_copy(gb1.at[j], t_s.at[jb1.at[j]], sems1,
                                 add=True)
            return c
        lax.fori_loop(0, half_t, _b, 0)
        _dr(sems1)

        @pl.when(sid == NS - 1)
        def _():
            _ld(dst_e, LAST - 1, ib0, semi0)
            _ld(src_e, LAST - 1, jb0, semi0)
            _dr(semi0)
            _dr(semi0)
            for j in range(CK):
                pltpu.async_copy(deg_s.at[ib0.at[j]], gb0.at[j], semg0)
            _dr(semg0)
            for j in range(CK):
                pltpu.async_copy(gb0.at[j], t_s.at[jb0.at[j]], sems0,
                                 add=True)
            _dr(sems0)
        plsc.subcore_barrier()

        pltpu.sync_copy(c_s.at[pl.ds(lo, SLICE)], degb)
        _rsqrt_inplace(degb)
        pltpu.sync_copy(t_s.at[pl.ds(lo, SLICE)], tb)

        def _m(i, c):
            s = pl.ds(i * 16, 16)
            degb[s] = degb[s] * tb[s]
            return c
        lax.fori_loop(0, NV, _m, 0)
        pltpu.sync_copy(degb, w_out.at[pl.ds(lo, SLICE)])
        plsc.subcore_barrier()

    @pl.when(cid == 0)
    def _():
        _run(d0, s0, w0)
        _run(d1, s1, w1)

    @pl.when(cid == 1)
    def _():
        _run(d2, s2, w2)
        _run(d3, s3, w3)


@jax.jit
def _sc_call(d0, d1, d2, d3, s0, s1, s2, s3):
    f = pl.kernel(
        _sc_body,
        out_type=tuple(
            jax.ShapeDtypeStruct((NPAD,), jnp.float32) for _ in range(4)),
        mesh=plsc.VectorSubcoreMesh(core_axis_name="c", subcore_axis_name="s"),
        scratch_types=[
            pltpu.VMEM((CK, LN), jnp.int32),
            pltpu.VMEM((CK, LN), jnp.int32),
            pltpu.VMEM((CK, LN), jnp.int32),
            pltpu.VMEM((CK, LN), jnp.int32),
            pltpu.VMEM((CK, LN), jnp.float32),
            pltpu.VMEM((CK, LN), jnp.float32),
            pltpu.VMEM((CKH, LN), jnp.int32),
            pltpu.VMEM((CKH, LN), jnp.int32),
            pltpu.VMEM((LN,), jnp.float32),
            pltpu.VMEM((SLICE,), jnp.float32),
            pltpu.VMEM((SLICE,), jnp.float32),
            pltpu.VMEM_SHARED((NPAD,), jnp.float32),
            pltpu.VMEM_SHARED((NPAD,), jnp.float32),
            pltpu.VMEM_SHARED((NPAD,), jnp.float32),
            pltpu.SemaphoreType.DMA,
            pltpu.SemaphoreType.DMA,
            pltpu.SemaphoreType.DMA,
            pltpu.SemaphoreType.DMA,
            pltpu.SemaphoreType.DMA,
            pltpu.SemaphoreType.DMA,
        ],
    )
    return f(d0, d1, d2, d3, s0, s1, s2, s3)


def _tc_body(w0, w1, w2, w3, xe, xb, xd, xw,
             wd, wdo, wt, ws_, bd, bdo, bt, bs, wfc, bfc, o_ref):
    bias = (bd[...] + bdo[...] + bt[...] + bs[...]).reshape(1, H)
    acc = jnp.zeros((1, H), jnp.float32)
    for wv, x, w in ((w0, xe, wd), (w1, xb, wdo), (w2, xd, wt), (w3, xw, ws_)):
        v = lax.dot_general(wv[...], x[...], (((1,), (1,)), ((), ())),
                            preferred_element_type=jnp.float32)
        acc = acc + jnp.dot(v, w[...], preferred_element_type=jnp.float32)
    hero_mean = acc * (1.0 / N) + bias
    o_ref[...] = (jnp.dot(hero_mean, wfc[...],
                          preferred_element_type=jnp.float32)
                  + bfc[...].reshape(1, OUT))


@jax.jit
def _tc_call(w0, w1, w2, w3, xe, xb, xd, xw,
             wd, wdo, wt, ws_, bd, bdo, bt, bs, wfc, bfc):
    return pl.pallas_call(
        _tc_body,
        out_shape=jax.ShapeDtypeStruct((1, OUT), jnp.float32),
    )(w0, w1, w2, w3, xe, xb, xd, xw, wd, wdo, wt, ws_, bd, bdo, bt, bs,
      wfc, bfc)


def _views(ei):
    return ei[1].reshape(RROWS, LN), ei[0].reshape(RROWS, LN)


def _pad_t(x):
    return jnp.pad(x.T, ((0, 0), (0, NPAD - N)))


def kernel(x_hero, x_enemy, x_bullet, x_door, x_wall,
           ei_defeats, ei_dodges, ei_to_go_to, ei_sees_block,
           W_defeats, b_defeats, W_dodges, b_dodges,
           W_to_go_to, b_to_go_to, W_sees_block, b_sees_block,
           W_fc, b_fc):
    d0, s0 = _views(ei_defeats)
    d1, s1 = _views(ei_dodges)
    d2, s2 = _views(ei_to_go_to)
    d3, s3 = _views(ei_sees_block)
    w0, w1, w2, w3 = _sc_call(d0, d1, d2, d3, s0, s1, s2, s3)
    return _tc_call(w0.reshape(1, NPAD), w1.reshape(1, NPAD),
                    w2.reshape(1, NPAD), w3.reshape(1, NPAD),
                    _pad_t(x_enemy), _pad_t(x_bullet),
                    _pad_t(x_door), _pad_t(x_wall),
                    W_defeats, W_dodges, W_to_go_to, W_sees_block,
                    b_defeats, b_dodges, b_to_go_to, b_sees_block,
                    W_fc, b_fc)
